# Initial kernel scaffold; baseline (speedup 1.0000x reference)
import jax
import jax.numpy as jnp
from jax.experimental import pallas as pl


def kernel(p, params):
    raise NotImplementedError("write your pallas kernel here")


# speedup vs baseline: 12.3057x; 12.3057x over previous
import functools

import jax
import jax.numpy as jnp
from jax import lax
from jax.experimental import pallas as pl
from jax.experimental.pallas import tpu as pltpu

_LAYERS = [dict(cin=0, cout=48, k=8, dil=1),
           dict(cin=48, cout=96, k=12, dil=2),
           dict(cin=96, cout=192, k=16, dil=2),
           dict(cin=192, cout=384, k=16, dil=3)]
_SAMPLES = [1024, 384, 128, 128]
_NEG_INF = -1e30
_BF = jnp.bfloat16
_F32 = jnp.float32


def _elu(v):
    return jnp.where(v > 0, v, jnp.exp(jnp.minimum(v, 0.0)) - 1.0)


def _bdot(a, b, dims=None):
    if dims is None:
        dims = (((1,), (0,)), ((), ()))
    return lax.dot_general(a.astype(_BF), b.astype(_BF), dims,
                           preferred_element_type=_F32)


def _lane_sum3(a):
    aa = a * a
    return aa[:, 0:1] + aa[:, 1:2] + aa[:, 2:3]


def _xconv_body(K, dil, cin, cd, cout, N, Mblk, *refs):
    if cin:
        (p_ref, pt_ref, q_ref, x_ref, w1_ref, b1_ref, w2_ref, b2_ref,
         wx_ref, bx_ref, wc_ref, out_ref) = refs
    else:
        (p_ref, pt_ref, q_ref, w1_ref, b1_ref, w2_ref, b2_ref,
         wx_ref, bx_ref, wc_ref, out_ref) = refs
    p = p_ref[0]
    pt = pt_ref[0]
    q = q_ref[0]
    if cin:
        table = jnp.concatenate([p, x_ref[0]], axis=1)
    else:
        table = p
    pt2 = pt * pt
    psq = pt2[0:1, :] + pt2[1:2, :] + pt2[2:3, :]
    qsq = _lane_sum3(q)
    qp = _bdot(q, pt)
    d2 = (qsq + psq) - 2.0 * qp
    neg = -d2
    col = lax.broadcasted_iota(jnp.int32, (Mblk, N), 1)
    sels = []
    for t in range(K * dil):
        mx = jnp.max(neg, axis=1, keepdims=True)
        idx = jnp.min(jnp.where(neg == mx, col, N), axis=1, keepdims=True)
        hit = col == idx
        if t % dil == 0:
            sels.append(jnp.dot(hit.astype(_F32), table,
                                preferred_element_type=_F32))
        if t < K * dil - 1:
            neg = jnp.where(hit, _NEG_INF, neg)

    w1 = w1_ref[...]
    b1 = b1_ref[...]
    w2 = w2_ref[...]
    b2 = b2_ref[...]
    prel = [s[:, :3] - q for s in sels]
    feats = []
    for k in range(K):
        h = _elu(_bdot(prel[k], w1) + b1)
        h = _elu(_bdot(h, w2) + b2)
        if cin:
            h = jnp.concatenate([h, sels[k][:, 3:]], axis=1)
        feats.append(h)
    pf = jnp.concatenate(prel, axis=1)
    xm = _bdot(pf, wx_ref[...]) + bx_ref[...]
    xmb = xm.astype(_BF).astype(_F32)
    fb = [f.astype(_BF).astype(_F32) for f in feats]
    c = cd + cin
    acc = jnp.zeros((Mblk, cout), _F32)
    for k in range(K):
        ft = jnp.zeros((Mblk, c), _F32)
        for j in range(K):
            ft = ft + xmb[:, k * K + j:k * K + j + 1] * fb[j]
        acc = acc + _bdot(ft, wc_ref[k * c:(k + 1) * c, :])
    out_ref[0] = acc


def _xconv(p, x, q, params, i, L, interpret=False):
    B, N, _ = p.shape
    M = q.shape[1]
    K, dil, cin, cout = L['k'], L['dil'], L['cin'], L['cout']
    cd = cout // 4
    c = cd + cin
    Mblk = 256 if M % 256 == 0 else M
    grid = (B, M // Mblk)
    body = functools.partial(_xconv_body, K, dil, cin, cd, cout, N, Mblk)
    wspec = lambda s: pl.BlockSpec(s, lambda b, j: (0,) * len(s))
    in_specs = [
        pl.BlockSpec((1, N, 3), lambda b, j: (b, 0, 0)),
        pl.BlockSpec((1, 3, N), lambda b, j: (b, 0, 0)),
        pl.BlockSpec((1, Mblk, 3), lambda b, j: (b, j, 0)),
    ]
    ins = [p, jnp.transpose(p, (0, 2, 1)), q]
    if cin:
        in_specs.append(pl.BlockSpec((1, N, cin), lambda b, j: (b, 0, 0)))
        ins.append(x)
    in_specs += [wspec((3, cd)), wspec((1, cd)), wspec((cd, cd)),
                 wspec((1, cd)), wspec((3 * K, K * K)), wspec((1, K * K)),
                 wspec((K * c, cout))]
    ins += [params['c%d_w1' % i], params['c%d_b1' % i].reshape(1, cd),
            params['c%d_w2' % i], params['c%d_b2' % i].reshape(1, cd),
            params['c%d_wx' % i], params['c%d_bx' % i].reshape(1, K * K),
            params['c%d_wc' % i]]
    out = pl.pallas_call(
        body,
        grid=grid,
        in_specs=in_specs,
        out_specs=pl.BlockSpec((1, Mblk, cout), lambda b, j: (b, j, 0)),
        out_shape=jax.ShapeDtypeStruct((B, M, cout), jnp.float32),
        interpret=interpret,
    )(*ins)
    return out


def _head_body(B, x_ref, w1_ref, g1_ref, b1_ref, w2_ref, g2_ref, b2_ref,
               fc_ref, fcb_ref, out_ref):
    tdims = (((1,), (1,)), ((), ()))
    outs = []
    for b in range(B):
        xb = x_ref[b]
        h = _bdot(xb, w1_ref[...], tdims)
        h = jnp.maximum(h * g1_ref[...] + b1_ref[...], 0.0)
        h = _bdot(h, w2_ref[...], tdims)
        h = jnp.maximum(h * g2_ref[...] + b2_ref[...], 0.0)
        o = _bdot(h, fc_ref[...], tdims) + fcb_ref[...]
        outs.append(jnp.mean(o, axis=0, keepdims=True))
    out_ref[...] = jnp.concatenate(outs, axis=0)


def _head(x, params, interpret=False):
    B, n, C = x.shape
    ncls = params['fc_w'].shape[0]
    body = functools.partial(_head_body, B)
    ins = [x, params['mlp_w1'], params['mlp_g1'].reshape(1, -1),
           params['mlp_b1'].reshape(1, -1), params['mlp_w2'],
           params['mlp_g2'].reshape(1, -1), params['mlp_b2'].reshape(1, -1),
           params['fc_w'], params['fc_b'].reshape(1, -1)]
    return pl.pallas_call(
        body,
        out_shape=jax.ShapeDtypeStruct((B, ncls), jnp.float32),
        interpret=interpret,
    )(*ins)


def _forward_impl(p, params, interpret=False):
    pts = p
    x = None
    for i, L in enumerate(_LAYERS):
        n = _SAMPLES[i]
        sidx = jax.random.permutation(jax.random.key(100 + i),
                                      pts.shape[1])[:n]
        q = jnp.take(pts, sidx, axis=1)
        x = _xconv(pts, x, q, params, i, L, interpret=interpret)
        pts = q
    return _head(x, params, interpret=interpret)


def kernel(p, params):
    return _forward_impl(p, params)
